# baseline (speedup 1.0000x reference)
import functools

import jax
import jax.numpy as jnp
from jax import lax
from jax.experimental import pallas as pl
from jax.experimental.pallas import tpu as pltpu
from jax.experimental.pallas import tpu_sc as plsc

N_NODES = 10000
D = 128

NC = 2
NS = 16
CHUNK = 128

N_PAD = 10112
ROWS_PER_TILE = N_PAD // NS
CROWS = 80


def _sc_aggregate(x, eye, idx_all, zacc, n_chunks):
    mesh = plsc.VectorSubcoreMesh(
        core_axis_name="c", subcore_axis_name="s", num_cores=NC, num_subcores=NS
    )

    @functools.partial(
        pl.kernel,
        out_type=[
            jax.ShapeDtypeStruct((NC, N_PAD, D), jnp.float32),
            jax.ShapeDtypeStruct((NC, CROWS, D), jnp.float32),
        ],
        mesh=mesh,
        scratch_types=[
            pltpu.VMEM((8, CHUNK), jnp.int32),
            pltpu.VMEM((CHUNK, D), jnp.float32),
            pltpu.VMEM((CHUNK, D), jnp.float32),
            pltpu.VMEM_SHARED((N_PAD, D), jnp.float32),
            pltpu.VMEM_SHARED((CROWS, D), jnp.float32),
            pltpu.VMEM_SHARED((D, D), jnp.float32),
            pltpu.SemaphoreType.DMA,
            pltpu.SemaphoreType.DMA,
            pltpu.SemaphoreType.DMA,
            pltpu.SemaphoreType.DMA,
        ],
    )
    def agg(x_hbm, eye_hbm, idx_hbm, zacc_hbm,
            sums_hbm, cnts_hbm,
            idp, bufa, bufb, acc_sh, cnt_sh, eye_sh,
            sema, semb, semea, semeb):
        c = lax.axis_index("c")
        s = lax.axis_index("s")
        row0 = s * ROWS_PER_TILE

        pltpu.sync_copy(zacc_hbm, bufa)
        for k in range(4):
            pltpu.sync_copy(bufa, acc_sh.at[pl.ds(row0 + k * CHUNK, CHUNK)])
        pltpu.sync_copy(bufa.at[pl.ds(0, 120)],
                        acc_sh.at[pl.ds(row0 + 4 * CHUNK, 120)])

        @pl.when(s == 0)
        def _():
            pltpu.sync_copy(bufa.at[pl.ds(0, CROWS)], cnt_sh)
            pltpu.sync_copy(eye_hbm, bufb)
            pltpu.sync_copy(bufb, eye_sh)

        plsc.subcore_barrier()

        def pair(k, carry):
            pltpu.sync_copy(idx_hbm.at[c, s, k], idp)
            cpa = pltpu.async_copy(x_hbm.at[idp.at[0]], bufa, sema)
            cpb = pltpu.async_copy(x_hbm.at[idp.at[4]], bufb, semb)
            cpa.wait()
            sca = pltpu.async_copy(bufa, acc_sh.at[idp.at[1]], semea, add=True)
            cpb.wait()
            scb = pltpu.async_copy(bufb, acc_sh.at[idp.at[5]], semeb, add=True)
            sca.wait()
            ega = pltpu.async_copy(eye_sh.at[idp.at[2]], bufa, semea)
            scb.wait()
            egb = pltpu.async_copy(eye_sh.at[idp.at[6]], bufb, semeb)
            ega.wait()
            pltpu.sync_copy(bufa, cnt_sh.at[idp.at[3]], add=True)
            egb.wait()
            pltpu.sync_copy(bufb, cnt_sh.at[idp.at[7]], add=True)
            return carry

        lax.fori_loop(0, n_chunks // 2, pair, 0)
        plsc.subcore_barrier()

        for k in range(4):
            sl = pl.ds(row0 + k * CHUNK, CHUNK)
            pltpu.sync_copy(acc_sh.at[sl], bufa)
            pltpu.sync_copy(bufa, sums_hbm.at[c, sl])
        sl = pl.ds(row0 + 4 * CHUNK, 120)
        pltpu.sync_copy(acc_sh.at[sl], bufa.at[pl.ds(0, 120)])
        pltpu.sync_copy(bufa.at[pl.ds(0, 120)], sums_hbm.at[c, sl])

        @pl.when(s == 0)
        def _():
            pltpu.sync_copy(cnt_sh, bufb.at[pl.ds(0, CROWS)])
            pltpu.sync_copy(bufb.at[pl.ds(0, CROWS)], cnts_hbm.at[c])

    return agg(x, eye, idx_all, zacc)


def _tc_finish_body(s0_ref, s1_ref, c0_ref, c1_ref, x_ref, w0_ref, w1_ref,
                    wl_ref, b_ref, out_ref):
    inv0 = 1.0 / jnp.maximum(c0_ref[...], 1.0)
    inv1 = 1.0 / jnp.maximum(c1_ref[...], 1.0)
    m0 = s0_ref[0] * inv0
    m1 = s1_ref[0] * inv1
    acc = jnp.dot(m0, w0_ref[...], preferred_element_type=jnp.float32)
    acc += jnp.dot(m1, w1_ref[...], preferred_element_type=jnp.float32)
    acc += jnp.dot(x_ref[...], wl_ref[...], preferred_element_type=jnp.float32)
    acc += b_ref[...]
    out_ref[...] = jnp.maximum(acc, 0.0)


def _tc_finish(sums, cnt0, cnt1, x, W_rel0, W_rel1, W_loop, b_loop):
    B = 1000
    grid = (N_NODES // B,)
    return pl.pallas_call(
        _tc_finish_body,
        grid=grid,
        in_specs=[
            pl.BlockSpec((1, B, D), lambda i: (0, i, 0)),
            pl.BlockSpec((1, B, D), lambda i: (1, i, 0)),
            pl.BlockSpec((B, 1), lambda i: (i, 0)),
            pl.BlockSpec((B, 1), lambda i: (i, 0)),
            pl.BlockSpec((B, D), lambda i: (i, 0)),
            pl.BlockSpec((D, D), lambda i: (0, 0)),
            pl.BlockSpec((D, D), lambda i: (0, 0)),
            pl.BlockSpec((D, D), lambda i: (0, 0)),
            pl.BlockSpec((1, D), lambda i: (0, 0)),
        ],
        out_specs=pl.BlockSpec((B, D), lambda i: (i, 0)),
        out_shape=jax.ShapeDtypeStruct((N_NODES, D), jnp.float32),
    )(sums, sums, cnt0, cnt1, x, W_rel0, W_rel1, W_loop, b_loop.reshape(1, D))


def kernel(x, edge_index_rel0, edge_index_rel1, W_rel0, W_rel1, W_loop, b_loop):
    n_edges = edge_index_rel0.shape[1]
    per_tile = -(-n_edges // (NS * 2 * CHUNK)) * 2 * CHUNK
    n_chunks = per_tile // CHUNK
    e_pad = per_tile * NS
    pad = e_pad - n_edges

    def prep(ei):
        src = ei[0].astype(jnp.int32)
        dst = ei[1].astype(jnp.int32)
        src = jnp.concatenate([src, jnp.zeros((pad,), jnp.int32)])
        dst = jnp.concatenate([dst, jnp.full((pad,), N_NODES, jnp.int32)])
        sh = (NS, n_chunks // 2, 2, CHUNK)
        packed = jnp.stack([src.reshape(sh), dst.reshape(sh),
                            (dst & 127).reshape(sh), (dst >> 7).reshape(sh)],
                           axis=3)
        return packed.reshape(NS, n_chunks // 2, 8, CHUNK)

    idx_all = jnp.stack([prep(edge_index_rel0), prep(edge_index_rel1)])

    zacc = jnp.zeros((CHUNK, D), jnp.float32)
    eye = jnp.eye(D, dtype=jnp.float32)

    sums, cnts = _sc_aggregate(x.astype(jnp.float32), eye, idx_all, zacc,
                               n_chunks)
    cnt_flat = cnts.reshape(NC, CROWS * D)[:, :N_NODES]
    cnt0 = cnt_flat[0].reshape(N_NODES, 1)
    cnt1 = cnt_flat[1].reshape(N_NODES, 1)
    return _tc_finish(sums, cnt0, cnt1, x, W_rel0, W_rel1, W_loop, b_loop)


# speedup vs baseline: 1.0539x; 1.0539x over previous
import functools

import jax
import jax.numpy as jnp
from jax import lax
from jax.experimental import pallas as pl
from jax.experimental.pallas import tpu as pltpu
from jax.experimental.pallas import tpu_sc as plsc

N_NODES = 10000
D = 128

NC = 2
NS = 16
CHUNK = 128

N_PAD = 10112
ROWS_PER_TILE = N_PAD // NS
CROWS = 80


def _sc_aggregate(x, eye, idx_all, zacc, n_chunks):
    mesh = plsc.VectorSubcoreMesh(
        core_axis_name="c", subcore_axis_name="s", num_cores=NC, num_subcores=NS
    )

    @functools.partial(
        pl.kernel,
        out_type=[
            jax.ShapeDtypeStruct((NC, N_PAD, D), jnp.float32),
            jax.ShapeDtypeStruct((NC, CROWS, D), jnp.float32),
        ],
        mesh=mesh,
        scratch_types=[
            pltpu.VMEM((8, CHUNK), jnp.int32),
            pltpu.VMEM((CHUNK, D), jnp.float32),
            pltpu.VMEM((CHUNK, D), jnp.float32),
            pltpu.VMEM_SHARED((N_PAD, D), jnp.float32),
            pltpu.VMEM_SHARED((CROWS, D), jnp.float32),
            pltpu.VMEM_SHARED((D, D), jnp.float32),
            pltpu.SemaphoreType.DMA,
            pltpu.SemaphoreType.DMA,
            pltpu.SemaphoreType.DMA,
            pltpu.SemaphoreType.DMA,
        ],
    )
    def agg(x_hbm, eye_hbm, idx_hbm, zacc_hbm,
            sums_hbm, cnts_hbm,
            idp, bufa, bufb, acc_sh, cnt_sh, eye_sh,
            sema, semb, semea, semeb):
        c = lax.axis_index("c")
        s = lax.axis_index("s")
        row0 = s * ROWS_PER_TILE

        pltpu.sync_copy(zacc_hbm, bufa)
        for k in range(4):
            pltpu.sync_copy(bufa, acc_sh.at[pl.ds(row0 + k * CHUNK, CHUNK)])
        pltpu.sync_copy(bufa.at[pl.ds(0, 120)],
                        acc_sh.at[pl.ds(row0 + 4 * CHUNK, 120)])

        @pl.when(s == 0)
        def _():
            pltpu.sync_copy(bufa.at[pl.ds(0, CROWS)], cnt_sh)
            pltpu.sync_copy(eye_hbm, bufb)
            pltpu.sync_copy(bufb, eye_sh)

        plsc.subcore_barrier()

        def pair(k, carry):
            pltpu.sync_copy(idx_hbm.at[c, s, k], idp)
            cpa = pltpu.async_copy(x_hbm.at[idp.at[0]], bufa, sema)
            cpb = pltpu.async_copy(x_hbm.at[idp.at[4]], bufb, semb)
            cpa.wait()
            pltpu.sync_copy(bufa, acc_sh.at[idp.at[1]], add=True)
            ega = pltpu.async_copy(eye_sh.at[idp.at[2]], bufa, semea)
            cpb.wait()
            pltpu.sync_copy(bufb, acc_sh.at[idp.at[5]], add=True)
            egb = pltpu.async_copy(eye_sh.at[idp.at[6]], bufb, semeb)
            ega.wait()
            pltpu.sync_copy(bufa, cnt_sh.at[idp.at[3]], add=True)
            egb.wait()
            pltpu.sync_copy(bufb, cnt_sh.at[idp.at[7]], add=True)
            return carry

        lax.fori_loop(0, n_chunks // 2, pair, 0)
        plsc.subcore_barrier()

        for k in range(4):
            sl = pl.ds(row0 + k * CHUNK, CHUNK)
            pltpu.sync_copy(acc_sh.at[sl], bufa)
            pltpu.sync_copy(bufa, sums_hbm.at[c, sl])
        sl = pl.ds(row0 + 4 * CHUNK, 120)
        pltpu.sync_copy(acc_sh.at[sl], bufa.at[pl.ds(0, 120)])
        pltpu.sync_copy(bufa.at[pl.ds(0, 120)], sums_hbm.at[c, sl])

        @pl.when(s == 0)
        def _():
            pltpu.sync_copy(cnt_sh, bufb.at[pl.ds(0, CROWS)])
            pltpu.sync_copy(bufb.at[pl.ds(0, CROWS)], cnts_hbm.at[c])

    return agg(x, eye, idx_all, zacc)


def _tc_finish_body(s0_ref, s1_ref, c0_ref, c1_ref, x_ref, w0_ref, w1_ref,
                    wl_ref, b_ref, out_ref):
    inv0 = 1.0 / jnp.maximum(c0_ref[...], 1.0)
    inv1 = 1.0 / jnp.maximum(c1_ref[...], 1.0)
    m0 = s0_ref[0] * inv0
    m1 = s1_ref[0] * inv1
    acc = jnp.dot(m0, w0_ref[...], preferred_element_type=jnp.float32)
    acc += jnp.dot(m1, w1_ref[...], preferred_element_type=jnp.float32)
    acc += jnp.dot(x_ref[...], wl_ref[...], preferred_element_type=jnp.float32)
    acc += b_ref[...]
    out_ref[...] = jnp.maximum(acc, 0.0)


def _tc_finish(sums, cnt0, cnt1, x, W_rel0, W_rel1, W_loop, b_loop):
    B = 1000
    grid = (N_NODES // B,)
    return pl.pallas_call(
        _tc_finish_body,
        grid=grid,
        in_specs=[
            pl.BlockSpec((1, B, D), lambda i: (0, i, 0)),
            pl.BlockSpec((1, B, D), lambda i: (1, i, 0)),
            pl.BlockSpec((B, 1), lambda i: (i, 0)),
            pl.BlockSpec((B, 1), lambda i: (i, 0)),
            pl.BlockSpec((B, D), lambda i: (i, 0)),
            pl.BlockSpec((D, D), lambda i: (0, 0)),
            pl.BlockSpec((D, D), lambda i: (0, 0)),
            pl.BlockSpec((D, D), lambda i: (0, 0)),
            pl.BlockSpec((1, D), lambda i: (0, 0)),
        ],
        out_specs=pl.BlockSpec((B, D), lambda i: (i, 0)),
        out_shape=jax.ShapeDtypeStruct((N_NODES, D), jnp.float32),
    )(sums, sums, cnt0, cnt1, x, W_rel0, W_rel1, W_loop, b_loop.reshape(1, D))


def kernel(x, edge_index_rel0, edge_index_rel1, W_rel0, W_rel1, W_loop, b_loop):
    n_edges = edge_index_rel0.shape[1]
    per_tile = -(-n_edges // (NS * 2 * CHUNK)) * 2 * CHUNK
    n_chunks = per_tile // CHUNK
    e_pad = per_tile * NS
    pad = e_pad - n_edges

    def prep(ei):
        src = ei[0].astype(jnp.int32)
        dst = ei[1].astype(jnp.int32)
        src = jnp.concatenate([src, jnp.zeros((pad,), jnp.int32)])
        dst = jnp.concatenate([dst, jnp.full((pad,), N_NODES, jnp.int32)])
        sh = (NS, n_chunks // 2, 2, CHUNK)
        packed = jnp.stack([src.reshape(sh), dst.reshape(sh),
                            (dst & 127).reshape(sh), (dst >> 7).reshape(sh)],
                           axis=3)
        return packed.reshape(NS, n_chunks // 2, 8, CHUNK)

    idx_all = jnp.stack([prep(edge_index_rel0), prep(edge_index_rel1)])

    zacc = jnp.zeros((CHUNK, D), jnp.float32)
    eye = jnp.eye(D, dtype=jnp.float32)

    sums, cnts = _sc_aggregate(x.astype(jnp.float32), eye, idx_all, zacc,
                               n_chunks)
    cnt_flat = cnts.reshape(NC, CROWS * D)[:, :N_NODES]
    cnt0 = cnt_flat[0].reshape(N_NODES, 1)
    cnt1 = cnt_flat[1].reshape(N_NODES, 1)
    return _tc_finish(sums, cnt0, cnt1, x, W_rel0, W_rel1, W_loop, b_loop)


# speedup vs baseline: 1.0613x; 1.0070x over previous
import functools

import jax
import jax.numpy as jnp
from jax import lax
from jax.experimental import pallas as pl
from jax.experimental.pallas import tpu as pltpu
from jax.experimental.pallas import tpu_sc as plsc

N_NODES = 10000
D = 128

NC = 2
NS = 16
CHUNK = 128

N_PAD = 10112
ROWS_PER_TILE = N_PAD // NS
CROWS = 80


def _sc_aggregate(x, eye, idx_all, zacc, n_chunks):
    mesh = plsc.VectorSubcoreMesh(
        core_axis_name="c", subcore_axis_name="s", num_cores=NC, num_subcores=NS
    )

    @functools.partial(
        pl.kernel,
        out_type=[
            jax.ShapeDtypeStruct((NC, N_PAD, D), jnp.float32),
            jax.ShapeDtypeStruct((NC, CROWS, D), jnp.float32),
        ],
        mesh=mesh,
        scratch_types=[
            pltpu.VMEM((4, CHUNK), jnp.int32),
            pltpu.VMEM((CHUNK,), jnp.int32),
            pltpu.VMEM((CHUNK,), jnp.int32),
            pltpu.VMEM((CHUNK,), jnp.int32),
            pltpu.VMEM((CHUNK,), jnp.int32),
            pltpu.VMEM((CHUNK, D), jnp.float32),
            pltpu.VMEM((CHUNK, D), jnp.float32),
            pltpu.VMEM_SHARED((N_PAD, D), jnp.float32),
            pltpu.VMEM_SHARED((CROWS, D), jnp.float32),
            pltpu.VMEM_SHARED((D, D), jnp.float32),
            pltpu.SemaphoreType.DMA,
            pltpu.SemaphoreType.DMA,
            pltpu.SemaphoreType.DMA,
            pltpu.SemaphoreType.DMA,
        ],
    )
    def agg(x_hbm, eye_hbm, idx_hbm, zacc_hbm,
            sums_hbm, cnts_hbm,
            idp, dloa, dhia, dlob, dhib, bufa, bufb, acc_sh, cnt_sh, eye_sh,
            sema, semb, semea, semeb):
        c = lax.axis_index("c")
        s = lax.axis_index("s")
        row0 = s * ROWS_PER_TILE

        pltpu.sync_copy(zacc_hbm, bufa)
        for k in range(4):
            pltpu.sync_copy(bufa, acc_sh.at[pl.ds(row0 + k * CHUNK, CHUNK)])
        pltpu.sync_copy(bufa.at[pl.ds(0, 120)],
                        acc_sh.at[pl.ds(row0 + 4 * CHUNK, 120)])

        @pl.when(s == 0)
        def _():
            pltpu.sync_copy(bufa.at[pl.ds(0, CROWS)], cnt_sh)
            pltpu.sync_copy(eye_hbm, bufb)
            pltpu.sync_copy(bufb, eye_sh)

        plsc.subcore_barrier()

        def pair(k, carry):
            pltpu.sync_copy(idx_hbm.at[c, s, k], idp)
            cpa = pltpu.async_copy(x_hbm.at[idp.at[0]], bufa, sema)
            cpb = pltpu.async_copy(x_hbm.at[idp.at[1]], bufb, semb)
            for t in range(CHUNK // 16):
                sl = pl.ds(t * 16, 16)
                da = idp[2, sl]
                dloa[sl] = da & 127
                dhia[sl] = da >> 7
                db = idp[3, sl]
                dlob[sl] = db & 127
                dhib[sl] = db >> 7
            cpa.wait()
            pltpu.sync_copy(bufa, acc_sh.at[idp.at[2]], add=True)
            ega = pltpu.async_copy(eye_sh.at[dloa], bufa, semea)
            cpb.wait()
            pltpu.sync_copy(bufb, acc_sh.at[idp.at[3]], add=True)
            egb = pltpu.async_copy(eye_sh.at[dlob], bufb, semeb)
            ega.wait()
            pltpu.sync_copy(bufa, cnt_sh.at[dhia], add=True)
            egb.wait()
            pltpu.sync_copy(bufb, cnt_sh.at[dhib], add=True)
            return carry

        lax.fori_loop(0, n_chunks // 2, pair, 0)
        plsc.subcore_barrier()

        for k in range(4):
            sl = pl.ds(row0 + k * CHUNK, CHUNK)
            pltpu.sync_copy(acc_sh.at[sl], bufa)
            pltpu.sync_copy(bufa, sums_hbm.at[c, sl])
        sl = pl.ds(row0 + 4 * CHUNK, 120)
        pltpu.sync_copy(acc_sh.at[sl], bufa.at[pl.ds(0, 120)])
        pltpu.sync_copy(bufa.at[pl.ds(0, 120)], sums_hbm.at[c, sl])

        @pl.when(s == 0)
        def _():
            pltpu.sync_copy(cnt_sh, bufb.at[pl.ds(0, CROWS)])
            pltpu.sync_copy(bufb.at[pl.ds(0, CROWS)], cnts_hbm.at[c])

    return agg(x, eye, idx_all, zacc)


def _tc_finish_body(s0_ref, s1_ref, c0_ref, c1_ref, x_ref, w0_ref, w1_ref,
                    wl_ref, b_ref, out_ref):
    inv0 = 1.0 / jnp.maximum(c0_ref[...], 1.0)
    inv1 = 1.0 / jnp.maximum(c1_ref[...], 1.0)
    m0 = s0_ref[0] * inv0
    m1 = s1_ref[0] * inv1
    acc = jnp.dot(m0, w0_ref[...], preferred_element_type=jnp.float32)
    acc += jnp.dot(m1, w1_ref[...], preferred_element_type=jnp.float32)
    acc += jnp.dot(x_ref[...], wl_ref[...], preferred_element_type=jnp.float32)
    acc += b_ref[...]
    out_ref[...] = jnp.maximum(acc, 0.0)


def _tc_finish(sums, cnt0, cnt1, x, W_rel0, W_rel1, W_loop, b_loop):
    B = 1000
    grid = (N_NODES // B,)
    return pl.pallas_call(
        _tc_finish_body,
        grid=grid,
        in_specs=[
            pl.BlockSpec((1, B, D), lambda i: (0, i, 0)),
            pl.BlockSpec((1, B, D), lambda i: (1, i, 0)),
            pl.BlockSpec((B, 1), lambda i: (i, 0)),
            pl.BlockSpec((B, 1), lambda i: (i, 0)),
            pl.BlockSpec((B, D), lambda i: (i, 0)),
            pl.BlockSpec((D, D), lambda i: (0, 0)),
            pl.BlockSpec((D, D), lambda i: (0, 0)),
            pl.BlockSpec((D, D), lambda i: (0, 0)),
            pl.BlockSpec((1, D), lambda i: (0, 0)),
        ],
        out_specs=pl.BlockSpec((B, D), lambda i: (i, 0)),
        out_shape=jax.ShapeDtypeStruct((N_NODES, D), jnp.float32),
    )(sums, sums, cnt0, cnt1, x, W_rel0, W_rel1, W_loop, b_loop.reshape(1, D))


def kernel(x, edge_index_rel0, edge_index_rel1, W_rel0, W_rel1, W_loop, b_loop):
    n_edges = edge_index_rel0.shape[1]
    per_tile = -(-n_edges // (NS * 2 * CHUNK)) * 2 * CHUNK
    n_chunks = per_tile // CHUNK
    e_pad = per_tile * NS
    pad = e_pad - n_edges

    def prep(ei):
        src = ei[0].astype(jnp.int32)
        dst = ei[1].astype(jnp.int32)
        src = jnp.concatenate([src, jnp.zeros((pad,), jnp.int32)])
        dst = jnp.concatenate([dst, jnp.full((pad,), N_NODES, jnp.int32)])
        packed = jnp.stack([src, dst]).reshape(2, NS, n_chunks // 2, 2, CHUNK)
        return packed.transpose(1, 2, 0, 3, 4).reshape(
            NS, n_chunks // 2, 4, CHUNK)

    idx_all = jnp.stack([prep(edge_index_rel0), prep(edge_index_rel1)])

    zacc = jnp.zeros((CHUNK, D), jnp.float32)
    eye = jnp.eye(D, dtype=jnp.float32)

    sums, cnts = _sc_aggregate(x.astype(jnp.float32), eye, idx_all, zacc,
                               n_chunks)
    cnt_flat = cnts.reshape(NC, CROWS * D)[:, :N_NODES]
    cnt0 = cnt_flat[0].reshape(N_NODES, 1)
    cnt1 = cnt_flat[1].reshape(N_NODES, 1)
    return _tc_finish(sums, cnt0, cnt1, x, W_rel0, W_rel1, W_loop, b_loop)
